# Initial kernel scaffold; baseline (speedup 1.0000x reference)
import jax
import jax.numpy as jnp
from jax.experimental import pallas as pl


def kernel(x, edge_index, batch_index, W1, b1, W2, b2, W3, b3):
    raise NotImplementedError("write your pallas kernel here")


# speedup vs baseline: 75.2461x; 75.2461x over previous
import functools

import jax
import jax.numpy as jnp
from jax import lax
from jax.experimental import pallas as pl
from jax.experimental.pallas import tpu as pltpu
from jax.experimental.pallas import tpu_sc as plsc

NN = 50000
NP = 51200
NE = 3200000
NG = 128
F = 16

NC = 2
NS = 16
NW = NC * NS

CH = 128
GRP = 8
EG = CH * GRP
NGROUPS = NE // EG
GFULL = NGROUPS // NW
GREM = NGROUPS % NW

RPT = NP // NS
RCH = 128
NRCH = RPT // RCH

DSPAN = NP // NS
DCH = 640
NDCH = DSPAN // DCH

PB = 1664
PC = 256
NPC = 7

TCR = 2048
TNB = NP // TCR

NEG_INF = float("-inf")


def _mesh():
    return plsc.VectorSubcoreMesh(core_axis_name="c", subcore_axis_name="s")


def _sc_deg(dst2d):
    @functools.partial(
        pl.kernel,
        mesh=_mesh(),
        compiler_params=pltpu.CompilerParams(use_tc_tiling_on_sc=False),
        out_type=jax.ShapeDtypeStruct((NC, NP), jnp.float32),
        scratch_types=[
            pltpu.VMEM_SHARED((NP,), jnp.float32),
            pltpu.VMEM((CH,), jnp.float32),
            pltpu.VMEM((DCH,), jnp.float32),
            pltpu.VMEM((GRP, CH), jnp.int32),
        ],
    )
    def k(dst_hbm, out_hbm, acc, ones_v, zer_v, dstv):
        c = lax.axis_index("c")
        s = lax.axis_index("s")
        w = s * NC + c

        def initbufs(i, carry):
            ones_v[pl.ds(i * 16, 16)] = jnp.ones((16,), jnp.float32)
            return carry
        lax.fori_loop(0, CH // 16, initbufs, None)

        def initz(i, carry):
            zer_v[pl.ds(i * 16, 16)] = jnp.zeros((16,), jnp.float32)
            return carry
        lax.fori_loop(0, DCH // 16, initz, None)

        base = s * DSPAN

        def zero_spmem(i, carry):
            pltpu.sync_copy(zer_v, acc.at[pl.ds(base + i * DCH, DCH)])
            return carry
        lax.fori_loop(0, NDCH, zero_spmem, None)

        plsc.subcore_barrier()

        def group(g):
            pltpu.sync_copy(dst_hbm.at[pl.ds(g * GRP, GRP)], dstv)
            for j in range(GRP):
                pltpu.sync_copy(ones_v, acc.at[dstv.at[j]], add=True)

        def body(gi, carry):
            group(w + gi * NW)
            return carry
        lax.fori_loop(0, GFULL, body, None)

        @pl.when(w < GREM)
        def _tail():
            group(w + GFULL * NW)

        plsc.subcore_barrier()

        def copy_out(i, carry):
            o = base + i * DCH
            pltpu.sync_copy(acc.at[pl.ds(o, DCH)], out_hbm.at[c].at[pl.ds(o, DCH)])
            return carry
        lax.fori_loop(0, NDCH, copy_out, None)

    return k(dst2d)


def _sc_layer(tab, src2d, dst2d):
    @functools.partial(
        pl.kernel,
        mesh=_mesh(),
        compiler_params=pltpu.CompilerParams(use_tc_tiling_on_sc=False),
        out_type=jax.ShapeDtypeStruct((NC, NP, F), jnp.float32),
        scratch_types=[
            pltpu.VMEM_SHARED((NP, F), jnp.float32),
            pltpu.VMEM((RCH, F), jnp.float32),
            pltpu.VMEM((GRP, CH), jnp.int32),
            pltpu.VMEM((GRP, CH), jnp.int32),
            pltpu.VMEM((GRP, CH, F), jnp.float32),
            pltpu.SemaphoreType.DMA,
        ],
    )
    def k(tab_hbm, src_hbm, dst_hbm, out_hbm, acc, zrow, srcv, dstv, rows, gsem):
        c = lax.axis_index("c")
        s = lax.axis_index("s")
        w = s * NC + c

        def initz(i, carry):
            zrow[i, :] = jnp.zeros((F,), jnp.float32)
            return carry
        lax.fori_loop(0, RCH, initz, None)

        rbase = s * RPT

        def zero_spmem(i, carry):
            pltpu.sync_copy(zrow, acc.at[pl.ds(rbase + i * RCH, RCH)])
            return carry
        lax.fori_loop(0, NRCH, zero_spmem, None)

        plsc.subcore_barrier()

        def group(g):
            pltpu.sync_copy(src_hbm.at[pl.ds(g * GRP, GRP)], srcv)
            pltpu.sync_copy(dst_hbm.at[pl.ds(g * GRP, GRP)], dstv)
            cps = [pltpu.async_copy(tab_hbm.at[srcv.at[j]], rows.at[j], gsem)
                   for j in range(GRP)]
            for cp in cps:
                cp.wait()
            for j in range(GRP):
                pltpu.sync_copy(rows.at[j], acc.at[dstv.at[j]], add=True)

        def body(gi, carry):
            group(w + gi * NW)
            return carry
        lax.fori_loop(0, GFULL, body, None)

        @pl.when(w < GREM)
        def _tail():
            group(w + GFULL * NW)

        plsc.subcore_barrier()

        def copy_out(i, carry):
            r0 = rbase + i * RCH
            pltpu.sync_copy(acc.at[pl.ds(r0, RCH)], out_hbm.at[c].at[pl.ds(r0, RCH)])
            return carry
        lax.fori_loop(0, NRCH, copy_out, None)

    return k(tab, src2d, dst2d)


def _sc_pool(h3f, batch):
    @functools.partial(
        pl.kernel,
        mesh=_mesh(),
        compiler_params=pltpu.CompilerParams(use_tc_tiling_on_sc=False),
        out_type=jax.ShapeDtypeStruct((NW, NG * F), jnp.float32),
        scratch_types=[
            pltpu.VMEM((NG * F,), jnp.float32),
            pltpu.VMEM((PC * F,), jnp.float32),
            pltpu.VMEM((PC,), jnp.int32),
        ],
    )
    def k(h_hbm, b_hbm, out_hbm, accf, hvf, bv):
        c = lax.axis_index("c")
        s = lax.axis_index("s")
        w = s * NC + c

        def ainit(i, carry):
            accf[pl.ds(i * F, F)] = jnp.full((F,), NEG_INF, jnp.float32)
            return carry
        lax.fori_loop(0, NG, ainit, None)

        def chunk(ci, carry):
            st = jnp.minimum(w * PB + ci * PC, NP - PC)
            pltpu.sync_copy(b_hbm.at[pl.ds(st, PC)], bv)
            pltpu.sync_copy(h_hbm.at[pl.ds(st * F, PC * F)], hvf)

            def node16(i16, carry2):
                gvec = bv[pl.ds(i16 * 16, 16)]
                for l in range(16):
                    g = gvec[l]
                    row = hvf[pl.ds((i16 * 16 + l) * F, F)]
                    o = g * F
                    cur = accf[pl.ds(o, F)]
                    accf[pl.ds(o, F)] = jnp.maximum(cur, row)
                return carry2
            lax.fori_loop(0, PC // 16, node16, None)
            return carry
        lax.fori_loop(0, NPC, chunk, None)

        pltpu.sync_copy(accf, out_hbm.at[w])

    return k(h3f, batch)


def _tc_pre1(degA, degB, x, W1):
    def body(dA, dB, xb, w1, ds_ref, xs_ref):
        ds = lax.rsqrt(dA[...] + dB[...] + 1.0)
        xw = jnp.dot(xb[...], w1[...], preferred_element_type=jnp.float32)
        ds_ref[...] = ds
        xs_ref[...] = xw * ds

    return pl.pallas_call(
        body,
        grid=(TNB,),
        in_specs=[
            pl.BlockSpec((TCR, 1), lambda i: (i, 0)),
            pl.BlockSpec((TCR, 1), lambda i: (i, 0)),
            pl.BlockSpec((TCR, 2), lambda i: (i, 0)),
            pl.BlockSpec((2, F), lambda i: (0, 0)),
        ],
        out_specs=[
            pl.BlockSpec((TCR, 1), lambda i: (i, 0)),
            pl.BlockSpec((TCR, F), lambda i: (i, 0)),
        ],
        out_shape=[
            jax.ShapeDtypeStruct((NP, 1), jnp.float32),
            jax.ShapeDtypeStruct((NP, F), jnp.float32),
        ],
    )(degA, degB, x, W1)


def _tc_mid(accA, accB, xs, ds, W, b):
    def body(aA, aB, xsb, dsb, wref, bref, out_ref):
        h = jnp.maximum(dsb[...] * (aA[...] + aB[...] + xsb[...]) + bref[...], 0.0)
        out_ref[...] = jnp.dot(h, wref[...], preferred_element_type=jnp.float32) * dsb[...]

    return pl.pallas_call(
        body,
        grid=(TNB,),
        in_specs=[
            pl.BlockSpec((TCR, F), lambda i: (i, 0)),
            pl.BlockSpec((TCR, F), lambda i: (i, 0)),
            pl.BlockSpec((TCR, F), lambda i: (i, 0)),
            pl.BlockSpec((TCR, 1), lambda i: (i, 0)),
            pl.BlockSpec((F, F), lambda i: (0, 0)),
            pl.BlockSpec((1, F), lambda i: (0, 0)),
        ],
        out_specs=pl.BlockSpec((TCR, F), lambda i: (i, 0)),
        out_shape=jax.ShapeDtypeStruct((NP, F), jnp.float32),
    )(accA, accB, xs, ds, W, b)


def _tc_post(accA, accB, xs, ds, b):
    def body(aA, aB, xsb, dsb, bref, out_ref):
        v = dsb[...] * (aA[...] + aB[...] + xsb[...]) + bref[...]
        row = (lax.broadcasted_iota(jnp.int32, (TCR, F), 0)
               + pl.program_id(0) * TCR)
        out_ref[...] = jnp.where(row < NN, v, NEG_INF)

    return pl.pallas_call(
        body,
        grid=(TNB,),
        in_specs=[
            pl.BlockSpec((TCR, F), lambda i: (i, 0)),
            pl.BlockSpec((TCR, F), lambda i: (i, 0)),
            pl.BlockSpec((TCR, F), lambda i: (i, 0)),
            pl.BlockSpec((TCR, 1), lambda i: (i, 0)),
            pl.BlockSpec((1, F), lambda i: (0, 0)),
        ],
        out_specs=pl.BlockSpec((TCR, F), lambda i: (i, 0)),
        out_shape=jax.ShapeDtypeStruct((NP, F), jnp.float32),
    )(accA, accB, xs, ds, b)


def _tc_final(pm):
    def body(pm_ref, out_ref):
        v = jnp.max(pm_ref[...], axis=0)
        col = lax.broadcasted_iota(jnp.int32, (NG, F), 1)
        valid = col < 5
        m = jnp.max(jnp.where(valid, v, NEG_INF), axis=1, keepdims=True)
        sdiff = v - m
        e = jnp.where(valid, jnp.exp(sdiff), 0.0)
        denom = jnp.sum(e, axis=1, keepdims=True)
        out_ref[...] = sdiff - jnp.log(denom)

    return pl.pallas_call(
        body,
        out_shape=jax.ShapeDtypeStruct((NG, F), jnp.float32),
    )(pm)


def kernel(x, edge_index, batch_index, W1, b1, W2, b2, W3, b3):
    src2d = edge_index[0].reshape(NE // CH, CH)
    dst2d = edge_index[1].reshape(NE // CH, CH)
    xp = jnp.pad(x, ((0, NP - NN), (0, 0)))
    bp = jnp.pad(batch_index, (0, NP - NN))

    deg = _sc_deg(dst2d)
    degA = deg[0].reshape(NP, 1)
    degB = deg[1].reshape(NP, 1)

    ds, xs1 = _tc_pre1(degA, degB, xp, W1)

    acc1 = _sc_layer(xs1, src2d, dst2d)
    xs2 = _tc_mid(acc1[0], acc1[1], xs1, ds, W2, b1.reshape(1, F))

    acc2 = _sc_layer(xs2, src2d, dst2d)
    W3p = jnp.zeros((F, F), jnp.float32).at[:, :5].set(W3)
    xs3 = _tc_mid(acc2[0], acc2[1], xs2, ds, W3p, b2.reshape(1, F))

    acc3 = _sc_layer(xs3, src2d, dst2d)
    b3p = jnp.zeros((1, F), jnp.float32).at[0, :5].set(b3)
    h3 = _tc_post(acc3[0], acc3[1], xs3, ds, b3p)

    pm = _sc_pool(h3.reshape(NP * F), bp)
    out = _tc_final(pm.reshape(NW, NG, F))
    return out[:, :5]
